# baseline (speedup 1.0000x reference)
import functools

import jax
import jax.numpy as jnp
from jax import lax
from jax.experimental import pallas as pl
from jax.experimental.pallas import tpu as pltpu
from jax.experimental.pallas import tpu_sc as plsc

NP_, NU_, NB_, NC_, NS_ = 50000, 20000, 2000, 500, 5000
N_ = NP_ + NU_ + NB_ + NC_ + NS_
NE_ = NU_ + NB_ + NC_ + NS_
NEPAD = 27504
H_, OUT_, DIN_ = 64, 32, 384

E_REAL = 470000
NMAC = 115
MAC = 256
EPT = NMAC * MAC
E_SIDE = 16 * EPT
ACC_ROWS = 50176
TRASH = 50000
ROWS_PT = ACC_ROWS // 16
ZROWS = 56
NM_ALL = 2 * 16 * NMAC

_MESH = plsc.VectorSubcoreMesh(
    core_axis_name="c", subcore_axis_name="s", num_cores=2, num_subcores=16)
_SDS = jax.ShapeDtypeStruct


def _zero_zbuf(zbuf):
    def zrow(i, _):
        zbuf[i, pl.ds(0, 16)] = jnp.zeros((16,), jnp.float32)
        zbuf[i, pl.ds(16, 16)] = jnp.zeros((16,), jnp.float32)
        return 0
    lax.fori_loop(0, ZROWS, zrow, 0)


def _zero_acc(acc, zbuf, s):
    def zblk(k, _):
        pltpu.sync_copy(zbuf, acc.at[pl.ds(s * ROWS_PT + k * ZROWS, ZROWS)])
        return 0
    lax.fori_loop(0, ROWS_PT // ZROWS, zblk, 0)


def _drain(acc, outP, outE, c, s):
    @pl.when((c == 0) & (s < 15))
    def _():
        pltpu.sync_copy(acc.at[pl.ds(s * 3128, 3128)],
                        outP.at[pl.ds(s * 3128, 3128)])

    @pl.when((c == 0) & (s == 15))
    def _():
        pltpu.sync_copy(acc.at[pl.ds(15 * 3128, 3080)],
                        outP.at[pl.ds(15 * 3128, 3080)])

    @pl.when((c == 1) & (s < 15))
    def _():
        pltpu.sync_copy(acc.at[pl.ds(s * 1720, 1720)],
                        outE.at[pl.ds(s * 1720, 1720)])

    @pl.when((c == 1) & (s == 15))
    def _():
        pltpu.sync_copy(acc.at[pl.ds(15 * 1720, 1704)],
                        outE.at[pl.ds(15 * 1720, 1704)])


def _make_segsum():
    scratch = [
        pltpu.VMEM((2, 2, 128), jnp.int32),
        pltpu.VMEM((2, 2, 128), jnp.int32),
        pltpu.VMEM((2, 2, 128), jnp.int32),
        pltpu.VMEM((2, 128, 32), jnp.float32),
        pltpu.VMEM((2, 128, 32), jnp.float32),
        pltpu.VMEM((2, 128, 32), jnp.float32),
        pltpu.VMEM_SHARED((ACC_ROWS, 32), jnp.float32),
        pltpu.VMEM((ZROWS, 32), jnp.float32),
        pltpu.SemaphoreType.DMA,
        pltpu.SemaphoreType.DMA,
        pltpu.SemaphoreType.DMA,
    ]

    def body(tableP, tableE, eidx, zd, outP, outE, mbuf0, mbuf1, mbuf2,
             rows0, rows1, rows2, acc, zbuf, isem, gsem, ssem):
        c = lax.axis_index("c")
        s = lax.axis_index("s")
        mbufs = (mbuf0, mbuf1, mbuf2)
        rows = (rows0, rows1, rows2)
        _zero_zbuf(zbuf)
        _zero_acc(acc, zbuf, s)
        plsc.subcore_barrier()
        mbase = (c * 16 + s) * NMAC
        pltpu.sync_copy(eidx.at[mbase], mbuf0)

        def step(g, p):
            mb, rw = mbufs[p], rows[p]
            omb, orw = mbufs[(p + 2) % 3], rows[(p + 2) % 3]

            @pl.when(g >= 2)
            def _():
                pltpu.make_async_copy(zd, rows[(p + 1) % 3], ssem).wait()

            @pl.when(g >= 1)
            def _():
                pltpu.make_async_copy(eidx.at[0], mb, isem).wait()

            @pl.when(g < NMAC - 1)
            def _():
                pltpu.async_copy(eidx.at[mbase + g + 1],
                                 mbufs[(p + 1) % 3], isem)

            @pl.when(c == 0)
            def _():
                for j in range(2):
                    pltpu.async_copy(tableE.at[mb.at[0, j]], rw.at[j], gsem)

            @pl.when(c == 1)
            def _():
                for j in range(2):
                    pltpu.async_copy(tableP.at[mb.at[0, j]], rw.at[j], gsem)

            @pl.when(g >= 1)
            def _():
                pltpu.make_async_copy(zd, orw, gsem).wait()
                for j in range(2):
                    pltpu.async_copy(orw.at[j], acc.at[omb.at[1, j]],
                                     ssem, add=True)

        def chunk(g, _):
            for p in range(3):
                @pl.when(g % 3 == p)
                def _(p=p):
                    step(g, p)
            return 0

        lax.fori_loop(0, NMAC, chunk, 0)
        pltpu.make_async_copy(zd, rows0, gsem).wait()
        for j in range(2):
            pltpu.async_copy(rows0.at[j], acc.at[mbuf0.at[1, j]], ssem,
                             add=True)
        pltpu.make_async_copy(zd, rows2, ssem).wait()
        pltpu.make_async_copy(zd, rows0, ssem).wait()
        plsc.subcore_barrier()
        _drain(acc, outP, outE, c, s)

    return pl.kernel(
        body,
        out_type=(_SDS((NP_, 32), jnp.float32),
                  _SDS((NEPAD, 32), jnp.float32)),
        mesh=_MESH,
        compiler_params=pltpu.CompilerParams(use_tc_tiling_on_sc=False),
        scratch_types=scratch,
    )


def _make_degree():
    scratch = [
        pltpu.VMEM((2, 128), jnp.int32),
        pltpu.VMEM((2, 128), jnp.int32),
        pltpu.VMEM((2, 128, 16), jnp.float32),
        pltpu.VMEM_SHARED((ACC_ROWS, 16), jnp.float32),
        pltpu.VMEM((ZROWS, 16), jnp.float32),
        pltpu.SemaphoreType.DMA,
    ]

    def body(eidx, zd16, outP, outE, dbufA, dbufB, ones16, acc, zbuf16,
             dsem):
        c = lax.axis_index("c")
        s = lax.axis_index("s")

        def zrow(i, _):
            zbuf16[i, pl.ds(0, 16)] = jnp.zeros((16,), jnp.float32)
            return 0
        lax.fori_loop(0, ZROWS, zrow, 0)

        def orow(i, _):
            ones16[i // 128, i % 128, pl.ds(0, 16)] = \
                jnp.ones((16,), jnp.float32)
            return 0
        lax.fori_loop(0, 256, orow, 0)
        _zero_acc(acc, zbuf16, s)
        plsc.subcore_barrier()
        mbase = (c * 16 + s) * NMAC

        def step(g, dbuf):
            @pl.when(g >= 2)
            def _():
                pltpu.make_async_copy(zd16, ones16, dsem).wait()

            pltpu.sync_copy(eidx.at[mbase + g, 1], dbuf)
            for j in range(2):
                pltpu.async_copy(ones16.at[j], acc.at[dbuf.at[j]], dsem,
                                 add=True)

        def chunk(g, _):
            @pl.when(g % 2 == 0)
            def _():
                step(g, dbufA)

            @pl.when(g % 2 == 1)
            def _():
                step(g, dbufB)
            return 0

        lax.fori_loop(0, NMAC, chunk, 0)
        pltpu.make_async_copy(zd16, ones16, dsem).wait()
        pltpu.make_async_copy(zd16, ones16, dsem).wait()
        plsc.subcore_barrier()
        _drain(acc, outP, outE, c, s)

    return pl.kernel(
        body,
        out_type=(_SDS((NP_, 16), jnp.float32),
                  _SDS((NEPAD, 16), jnp.float32)),
        mesh=_MESH,
        compiler_params=pltpu.CompilerParams(use_tc_tiling_on_sc=False),
        scratch_types=scratch,
    )


_segsum = _make_segsum()
_sc_degree = _make_degree()


def _k1_body(xp, wp, bp, out):
    out[...] = jax.nn.relu(
        jnp.dot(xp[...], wp[...], preferred_element_type=jnp.float32)
        + bp[...])


def _tc_prod(x_product, Wp, bp):
    R, G = 2000, 25
    return pl.pallas_call(
        _k1_body,
        grid=(G,),
        in_specs=[
            pl.BlockSpec((R, DIN_), lambda i: (i, 0)),
            pl.BlockSpec((DIN_, H_), lambda i: (0, 0)),
            pl.BlockSpec((1, H_), lambda i: (0, 0)),
        ],
        out_specs=pl.BlockSpec((R, H_), lambda i: (i, 0)),
        out_shape=_SDS((NP_, H_), jnp.float32),
    )(x_product, Wp, bp.reshape(1, H_))


def _k2_body(x, w1l, w1r, b1, outa, outb, outr):
    xb = x[...]
    xl = jnp.dot(xb, w1l[...], preferred_element_type=jnp.float32)
    outa[...] = xl[:, :32]
    outb[...] = xl[:, 32:]
    outr[...] = jnp.dot(xb, w1r[...], preferred_element_type=jnp.float32) \
        + b1[...]


def _tc_lin1(x, W1l, W1r, b1, R, G):
    n = R * G
    return pl.pallas_call(
        _k2_body,
        grid=(G,),
        in_specs=[
            pl.BlockSpec((R, H_), lambda i: (i, 0)),
            pl.BlockSpec((H_, H_), lambda i: (0, 0)),
            pl.BlockSpec((H_, H_), lambda i: (0, 0)),
            pl.BlockSpec((1, H_), lambda i: (0, 0)),
        ],
        out_specs=[
            pl.BlockSpec((R, 32), lambda i: (i, 0)),
            pl.BlockSpec((R, 32), lambda i: (i, 0)),
            pl.BlockSpec((R, H_), lambda i: (i, 0)),
        ],
        out_shape=[
            _SDS((n, 32), jnp.float32),
            _SDS((n, 32), jnp.float32),
            _SDS((n, H_), jnp.float32),
        ],
    )(x, W1l, W1r, b1.reshape(1, H_))


def _k3_body(agga, aggb, deg, xrb, w2l, w2r, b2, outl, outr):
    inv = 1.0 / jnp.maximum(deg[...][:, :1], 1.0)
    h = jax.nn.relu(
        jnp.concatenate([agga[...] * inv, aggb[...] * inv], axis=1)
        + xrb[...])
    outl[...] = jnp.dot(h, w2l[...], preferred_element_type=jnp.float32)
    outr[...] = jnp.dot(h, w2r[...], preferred_element_type=jnp.float32) \
        + b2[...]


def _tc_layer2in(aggA, aggB, deg16, xrb, W2l, W2r, b2, R, G):
    n = R * G
    return pl.pallas_call(
        _k3_body,
        grid=(G,),
        in_specs=[
            pl.BlockSpec((R, 32), lambda i: (i, 0)),
            pl.BlockSpec((R, 32), lambda i: (i, 0)),
            pl.BlockSpec((R, 16), lambda i: (i, 0)),
            pl.BlockSpec((R, H_), lambda i: (i, 0)),
            pl.BlockSpec((H_, OUT_), lambda i: (0, 0)),
            pl.BlockSpec((H_, OUT_), lambda i: (0, 0)),
            pl.BlockSpec((1, OUT_), lambda i: (0, 0)),
        ],
        out_specs=[
            pl.BlockSpec((R, OUT_), lambda i: (i, 0)),
            pl.BlockSpec((R, OUT_), lambda i: (i, 0)),
        ],
        out_shape=[
            _SDS((n, OUT_), jnp.float32),
            _SDS((n, OUT_), jnp.float32),
        ],
    )(aggA, aggB, deg16, xrb, W2l, W2r, b2.reshape(1, OUT_))


def _k4_body(agg2, deg, hrb, out):
    inv = 1.0 / jnp.maximum(deg[...][:, :1], 1.0)
    out[...] = agg2[...] * inv + hrb[...]


def _tc_final(agg2, deg16, hrb2, R, G):
    return pl.pallas_call(
        _k4_body,
        grid=(G,),
        in_specs=[
            pl.BlockSpec((R, OUT_), lambda i: (i, 0)),
            pl.BlockSpec((R, 16), lambda i: (i, 0)),
            pl.BlockSpec((R, OUT_), lambda i: (i, 0)),
        ],
        out_specs=pl.BlockSpec((R, OUT_), lambda i: (i, 0)),
        out_shape=_SDS((R * G, OUT_), jnp.float32),
    )(agg2, deg16, hrb2)


def _edges(edge_pb, edge_pc, edge_ps, edge_up):
    i32 = jnp.int32
    npad = E_SIDE - E_REAL
    padz = jnp.zeros((npad,), i32)
    padt = jnp.full((npad,), TRASH, i32)
    srcA = jnp.concatenate([
        edge_pb[1] + NU_, edge_pc[1] + (NU_ + NB_),
        edge_ps[1] + (NU_ + NB_ + NC_), edge_up[0], padz])
    dstA = jnp.concatenate([
        edge_pb[0], edge_pc[0], edge_ps[0], edge_up[1], padt])
    srcB = jnp.concatenate([
        edge_pb[0], edge_pc[0], edge_ps[0], edge_up[1], padz])
    dstB = jnp.concatenate([
        edge_pb[1] + NU_, edge_pc[1] + (NU_ + NB_),
        edge_ps[1] + (NU_ + NB_ + NC_), edge_up[0], padt])
    sr = jnp.concatenate([srcA, srcB]).astype(i32).reshape(2, 16, NMAC, 2, 128)
    dr = jnp.concatenate([dstA, dstB]).astype(i32).reshape(2, 16, NMAC, 2, 128)
    return jnp.stack([sr, dr], axis=3).reshape(NM_ALL, 2, 2, 128)


def kernel(x_product, edge_pb, edge_pc, edge_ps, edge_up, user_emb,
           brand_emb, category_emb, shop_emb, Wp, bp, W1l, W1r, b1,
           W2l, W2r, b2):
    eidx = _edges(edge_pb, edge_pc, edge_ps, edge_up)
    zd = jnp.zeros((2, 128, 32), jnp.float32)
    zd16 = jnp.zeros((2, 128, 16), jnp.float32)
    degP, degE = _sc_degree(eidx, zd16)

    xE = jnp.concatenate([user_emb, brand_emb, category_emb, shop_emb,
                          jnp.zeros((NEPAD - NE_, H_), jnp.float32)], axis=0)
    xlAE, xlBE, xrbE = _tc_lin1(xE, W1l, W1r, b1, 3056, 9)

    prod = _tc_prod(x_product, Wp, bp)
    xlAP, xlBP, xrbP = _tc_lin1(prod, W1l, W1r, b1, 2000, 25)

    aggAP, aggAE = _segsum(xlAP, xlAE, eidx, zd)
    aggBP, aggBE = _segsum(xlBP, xlBE, eidx, zd)
    hlP, hrbP = _tc_layer2in(aggAP, aggBP, degP, xrbP, W2l, W2r, b2,
                             2000, 25)
    hlE, hrbE = _tc_layer2in(aggAE, aggBE, degE, xrbE, W2l, W2r, b2,
                             3056, 9)
    agg2P, agg2E = _segsum(hlP, hlE, eidx, zd)
    outP = _tc_final(agg2P, degP, hrbP, 2000, 25)
    outE = _tc_final(agg2E, degE, hrbE, 3056, 9)
    return (outP, outE[:NU_], outE[NU_:NU_ + NB_],
            outE[NU_ + NB_:NU_ + NB_ + NC_],
            outE[NU_ + NB_ + NC_:NE_])


# speedup vs baseline: 1.0569x; 1.0569x over previous
import functools

import jax
import jax.numpy as jnp
from jax import lax
from jax.experimental import pallas as pl
from jax.experimental.pallas import tpu as pltpu
from jax.experimental.pallas import tpu_sc as plsc

NP_, NU_, NB_, NC_, NS_ = 50000, 20000, 2000, 500, 5000
N_ = NP_ + NU_ + NB_ + NC_ + NS_
NE_ = NU_ + NB_ + NC_ + NS_
NEPAD = 27504
H_, OUT_, DIN_ = 64, 32, 384

E_REAL = 470000
NMAC = 115
MAC = 256
EPT = NMAC * MAC
E_SIDE = 16 * EPT
ACC_ROWS = 50176
TRASH = 50000
ROWS_PT = ACC_ROWS // 16
ZROWS = 56
NM_ALL = 2 * 16 * NMAC

_MESH = plsc.VectorSubcoreMesh(
    core_axis_name="c", subcore_axis_name="s", num_cores=2, num_subcores=16)
_SDS = jax.ShapeDtypeStruct


def _zero_zbuf(zbuf):
    def zrow(i, _):
        zbuf[i, pl.ds(0, 16)] = jnp.zeros((16,), jnp.float32)
        zbuf[i, pl.ds(16, 16)] = jnp.zeros((16,), jnp.float32)
        return 0
    lax.fori_loop(0, ZROWS, zrow, 0)


def _zero_acc(acc, zbuf, s):
    def zblk(k, _):
        pltpu.sync_copy(zbuf, acc.at[pl.ds(s * ROWS_PT + k * ZROWS, ZROWS)])
        return 0
    lax.fori_loop(0, ROWS_PT // ZROWS, zblk, 0)


def _drain(acc, outP, outE, c, s):
    @pl.when((c == 0) & (s < 15))
    def _():
        pltpu.sync_copy(acc.at[pl.ds(s * 3128, 3128)],
                        outP.at[pl.ds(s * 3128, 3128)])

    @pl.when((c == 0) & (s == 15))
    def _():
        pltpu.sync_copy(acc.at[pl.ds(15 * 3128, 3080)],
                        outP.at[pl.ds(15 * 3128, 3080)])

    @pl.when((c == 1) & (s < 15))
    def _():
        pltpu.sync_copy(acc.at[pl.ds(s * 1720, 1720)],
                        outE.at[pl.ds(s * 1720, 1720)])

    @pl.when((c == 1) & (s == 15))
    def _():
        pltpu.sync_copy(acc.at[pl.ds(15 * 1720, 1704)],
                        outE.at[pl.ds(15 * 1720, 1704)])


def _make_segsum():
    scratch = [
        pltpu.VMEM((2, 2, 128), jnp.int32),
        pltpu.VMEM((2, 2, 128), jnp.int32),
        pltpu.VMEM((2, 2, 128), jnp.int32),
        pltpu.VMEM((2, 128, 32), jnp.float32),
        pltpu.VMEM((2, 128, 32), jnp.float32),
        pltpu.VMEM((2, 128, 32), jnp.float32),
        pltpu.VMEM_SHARED((ACC_ROWS, 32), jnp.float32),
        pltpu.VMEM((ZROWS, 32), jnp.float32),
        pltpu.SemaphoreType.DMA,
        pltpu.SemaphoreType.DMA,
        pltpu.SemaphoreType.DMA,
    ]

    def body(tableP, tableE, eidx, zd, outP, outE, mbuf0, mbuf1, mbuf2,
             rows0, rows1, rows2, acc, zbuf, isem, gsem, ssem):
        c = lax.axis_index("c")
        s = lax.axis_index("s")
        mbufs = (mbuf0, mbuf1, mbuf2)
        rows = (rows0, rows1, rows2)
        _zero_zbuf(zbuf)
        _zero_acc(acc, zbuf, s)
        plsc.subcore_barrier()
        mbase = (c * 16 + s) * NMAC
        pltpu.sync_copy(eidx.at[mbase], mbuf0)

        def step(g, p):
            mb, rw = mbufs[p], rows[p]
            omb, orw = mbufs[(p + 2) % 3], rows[(p + 2) % 3]

            @pl.when(g >= 2)
            def _():
                pltpu.make_async_copy(zd, rows[(p + 1) % 3], ssem).wait()

            @pl.when(g >= 1)
            def _():
                pltpu.make_async_copy(eidx.at[0], mb, isem).wait()

            @pl.when(g < NMAC - 1)
            def _():
                pltpu.async_copy(eidx.at[mbase + g + 1],
                                 mbufs[(p + 1) % 3], isem)

            @pl.when(c == 0)
            def _():
                for j in range(2):
                    pltpu.async_copy(tableE.at[mb.at[0, j]], rw.at[j], gsem)

            @pl.when(c == 1)
            def _():
                for j in range(2):
                    pltpu.async_copy(tableP.at[mb.at[0, j]], rw.at[j], gsem)

            @pl.when(g >= 1)
            def _():
                pltpu.make_async_copy(zd, orw, gsem).wait()
                for j in range(2):
                    pltpu.async_copy(orw.at[j], acc.at[omb.at[1, j]],
                                     ssem, add=True)

        def chunk(g, _):
            for p in range(3):
                @pl.when(g % 3 == p)
                def _(p=p):
                    step(g, p)
            return 0

        lax.fori_loop(0, NMAC, chunk, 0)
        pltpu.make_async_copy(zd, rows0, gsem).wait()
        for j in range(2):
            pltpu.async_copy(rows0.at[j], acc.at[mbuf0.at[1, j]], ssem,
                             add=True)
        pltpu.make_async_copy(zd, rows2, ssem).wait()
        pltpu.make_async_copy(zd, rows0, ssem).wait()
        plsc.subcore_barrier()
        _drain(acc, outP, outE, c, s)

    return pl.kernel(
        body,
        out_type=(_SDS((NP_, 32), jnp.float32),
                  _SDS((NEPAD, 32), jnp.float32)),
        mesh=_MESH,
        compiler_params=pltpu.CompilerParams(use_tc_tiling_on_sc=False),
        scratch_types=scratch,
    )


def _make_degree():
    scratch = [
        pltpu.VMEM((2, 128), jnp.int32),
        pltpu.VMEM((2, 128), jnp.int32),
        pltpu.VMEM((2, 128, 16), jnp.float32),
        pltpu.VMEM_SHARED((ACC_ROWS, 16), jnp.float32),
        pltpu.VMEM((ZROWS, 16), jnp.float32),
        pltpu.SemaphoreType.DMA,
    ]

    def body(eidx, zd16, outP, outE, dbufA, dbufB, ones16, acc, zbuf16,
             dsem):
        c = lax.axis_index("c")
        s = lax.axis_index("s")

        def zrow(i, _):
            zbuf16[i, pl.ds(0, 16)] = jnp.zeros((16,), jnp.float32)
            return 0
        lax.fori_loop(0, ZROWS, zrow, 0)

        def orow(i, _):
            ones16[i // 128, i % 128, pl.ds(0, 16)] = \
                jnp.ones((16,), jnp.float32)
            return 0
        lax.fori_loop(0, 256, orow, 0)
        _zero_acc(acc, zbuf16, s)
        plsc.subcore_barrier()
        mbase = (c * 16 + s) * NMAC

        def step(g, dbuf):
            @pl.when(g >= 2)
            def _():
                pltpu.make_async_copy(zd16, ones16, dsem).wait()

            pltpu.sync_copy(eidx.at[mbase + g, 1], dbuf)
            for j in range(2):
                pltpu.async_copy(ones16.at[j], acc.at[dbuf.at[j]], dsem,
                                 add=True)

        def chunk(g, _):
            @pl.when(g % 2 == 0)
            def _():
                step(g, dbufA)

            @pl.when(g % 2 == 1)
            def _():
                step(g, dbufB)
            return 0

        lax.fori_loop(0, NMAC, chunk, 0)
        pltpu.make_async_copy(zd16, ones16, dsem).wait()
        pltpu.make_async_copy(zd16, ones16, dsem).wait()
        plsc.subcore_barrier()
        _drain(acc, outP, outE, c, s)

    return pl.kernel(
        body,
        out_type=(_SDS((NP_, 16), jnp.float32),
                  _SDS((NEPAD, 16), jnp.float32)),
        mesh=_MESH,
        compiler_params=pltpu.CompilerParams(use_tc_tiling_on_sc=False),
        scratch_types=scratch,
    )


_segsum = _make_segsum()
_sc_degree = _make_degree()


def _k1_body(xp, wp, bp, out):
    out[...] = jax.nn.relu(
        jnp.dot(xp[...], wp[...], preferred_element_type=jnp.float32)
        + bp[...])


def _tc_prod(x_product, Wp, bp):
    R, G = 2000, 25
    return pl.pallas_call(
        _k1_body,
        grid=(G,),
        in_specs=[
            pl.BlockSpec((R, DIN_), lambda i: (i, 0)),
            pl.BlockSpec((DIN_, H_), lambda i: (0, 0)),
            pl.BlockSpec((1, H_), lambda i: (0, 0)),
        ],
        out_specs=pl.BlockSpec((R, H_), lambda i: (i, 0)),
        out_shape=_SDS((NP_, H_), jnp.float32),
    )(x_product, Wp, bp.reshape(1, H_))


def _k2_body(x, w1l, w1r, b1, outa, outb, outr):
    xb = x[...]
    xl = jnp.dot(xb, w1l[...], preferred_element_type=jnp.float32)
    outa[...] = xl[:, :32]
    outb[...] = xl[:, 32:]
    outr[...] = jnp.dot(xb, w1r[...], preferred_element_type=jnp.float32) \
        + b1[...]


def _tc_lin1(x, W1l, W1r, b1, R, G):
    n = R * G
    return pl.pallas_call(
        _k2_body,
        grid=(G,),
        in_specs=[
            pl.BlockSpec((R, H_), lambda i: (i, 0)),
            pl.BlockSpec((H_, H_), lambda i: (0, 0)),
            pl.BlockSpec((H_, H_), lambda i: (0, 0)),
            pl.BlockSpec((1, H_), lambda i: (0, 0)),
        ],
        out_specs=[
            pl.BlockSpec((R, 32), lambda i: (i, 0)),
            pl.BlockSpec((R, 32), lambda i: (i, 0)),
            pl.BlockSpec((R, H_), lambda i: (i, 0)),
        ],
        out_shape=[
            _SDS((n, 32), jnp.float32),
            _SDS((n, 32), jnp.float32),
            _SDS((n, H_), jnp.float32),
        ],
    )(x, W1l, W1r, b1.reshape(1, H_))


def _k3_body(agga, aggb, deg, xrb, w2l, w2r, b2, outl, outr):
    inv = 1.0 / jnp.maximum(deg[...][:, :1], 1.0)
    h = jax.nn.relu(
        jnp.concatenate([agga[...] * inv, aggb[...] * inv], axis=1)
        + xrb[...])
    outl[...] = jnp.dot(h, w2l[...], preferred_element_type=jnp.float32)
    outr[...] = jnp.dot(h, w2r[...], preferred_element_type=jnp.float32) \
        + b2[...]


def _tc_layer2in(aggA, aggB, deg16, xrb, W2l, W2r, b2, R, G):
    n = R * G
    return pl.pallas_call(
        _k3_body,
        grid=(G,),
        in_specs=[
            pl.BlockSpec((R, 32), lambda i: (i, 0)),
            pl.BlockSpec((R, 32), lambda i: (i, 0)),
            pl.BlockSpec((R, 16), lambda i: (i, 0)),
            pl.BlockSpec((R, H_), lambda i: (i, 0)),
            pl.BlockSpec((H_, OUT_), lambda i: (0, 0)),
            pl.BlockSpec((H_, OUT_), lambda i: (0, 0)),
            pl.BlockSpec((1, OUT_), lambda i: (0, 0)),
        ],
        out_specs=[
            pl.BlockSpec((R, OUT_), lambda i: (i, 0)),
            pl.BlockSpec((R, OUT_), lambda i: (i, 0)),
        ],
        out_shape=[
            _SDS((n, OUT_), jnp.float32),
            _SDS((n, OUT_), jnp.float32),
        ],
    )(aggA, aggB, deg16, xrb, W2l, W2r, b2.reshape(1, OUT_))


def _k4_body(agg2, deg, hrb, out):
    inv = 1.0 / jnp.maximum(deg[...][:, :1], 1.0)
    out[...] = agg2[...] * inv + hrb[...]


def _tc_final(agg2, deg16, hrb2, R, G):
    return pl.pallas_call(
        _k4_body,
        grid=(G,),
        in_specs=[
            pl.BlockSpec((R, OUT_), lambda i: (i, 0)),
            pl.BlockSpec((R, 16), lambda i: (i, 0)),
            pl.BlockSpec((R, OUT_), lambda i: (i, 0)),
        ],
        out_specs=pl.BlockSpec((R, OUT_), lambda i: (i, 0)),
        out_shape=_SDS((R * G, OUT_), jnp.float32),
    )(agg2, deg16, hrb2)


def _edges(edge_pb, edge_pc, edge_ps, edge_up):
    i32 = jnp.int32
    npad = E_SIDE - E_REAL
    padz = jnp.zeros((npad,), i32)
    padt = jnp.full((npad,), TRASH, i32)
    srcA = jnp.concatenate([
        edge_pb[1] + NU_, edge_pc[1] + (NU_ + NB_),
        edge_ps[1] + (NU_ + NB_ + NC_), edge_up[0], padz])
    dstA = jnp.concatenate([
        edge_pb[0], edge_pc[0], edge_ps[0], edge_up[1], padt])
    srcB = jnp.concatenate([
        edge_pb[0], edge_pc[0], edge_ps[0], edge_up[1], padz])
    dstB = jnp.concatenate([
        edge_pb[1] + NU_, edge_pc[1] + (NU_ + NB_),
        edge_ps[1] + (NU_ + NB_ + NC_), edge_up[0], padt])
    sr = jnp.concatenate([srcA, srcB]).astype(i32).reshape(2, 16, NMAC, 2, 128)
    dr = jnp.concatenate([dstA, dstB]).astype(i32).reshape(2, 16, NMAC, 2, 128)
    return jnp.stack([sr, dr], axis=3).reshape(NM_ALL, 2, 2, 128)


def kernel(x_product, edge_pb, edge_pc, edge_ps, edge_up, user_emb,
           brand_emb, category_emb, shop_emb, Wp, bp, W1l, W1r, b1,
           W2l, W2r, b2):
    eidx = _edges(edge_pb, edge_pc, edge_ps, edge_up)
    (eidx, x_product, user_emb, brand_emb, category_emb, shop_emb) = \
        lax.optimization_barrier(
            (eidx, x_product, user_emb, brand_emb, category_emb, shop_emb))
    zd = jnp.zeros((2, 128, 32), jnp.float32)
    zd16 = jnp.zeros((2, 128, 16), jnp.float32)
    degP, degE = _sc_degree(eidx, zd16)
    degP, degE, zd = lax.optimization_barrier((degP, degE, zd))

    xE = jnp.concatenate([user_emb, brand_emb, category_emb, shop_emb,
                          jnp.zeros((NEPAD - NE_, H_), jnp.float32)], axis=0)
    xlAE, xlBE, xrbE = _tc_lin1(xE, W1l, W1r, b1, 3056, 9)

    prod = _tc_prod(x_product, Wp, bp)
    xlAP, xlBP, xrbP = _tc_lin1(prod, W1l, W1r, b1, 2000, 25)

    aggAP, aggAE = _segsum(xlAP, xlAE, eidx, zd)
    aggBP, aggBE = _segsum(xlBP, xlBE, eidx, zd)
    hlP, hrbP = _tc_layer2in(aggAP, aggBP, degP, xrbP, W2l, W2r, b2,
                             2000, 25)
    hlE, hrbE = _tc_layer2in(aggAE, aggBE, degE, xrbE, W2l, W2r, b2,
                             3056, 9)
    agg2P, agg2E = _segsum(hlP, hlE, eidx, zd)
    outP = _tc_final(agg2P, degP, hrbP, 2000, 25)
    outE = _tc_final(agg2E, degE, hrbE, 3056, 9)
    return (outP, outE[:NU_], outE[NU_:NU_ + NB_],
            outE[NU_ + NB_:NU_ + NB_ + NC_],
            outE[NU_ + NB_ + NC_:NE_])


# speedup vs baseline: 1.0781x; 1.0201x over previous
import functools

import jax
import jax.numpy as jnp
from jax import lax
from jax.experimental import pallas as pl
from jax.experimental.pallas import tpu as pltpu
from jax.experimental.pallas import tpu_sc as plsc

NP_, NU_, NB_, NC_, NS_ = 50000, 20000, 2000, 500, 5000
N_ = NP_ + NU_ + NB_ + NC_ + NS_
NE_ = NU_ + NB_ + NC_ + NS_
NEPAD = 27504
H_, OUT_, DIN_ = 64, 32, 384

E_REAL = 470000
NMAC = 115
MAC = 256
EPT = NMAC * MAC
E_SIDE = 16 * EPT
ACC_ROWS = 50176
TRASH = 50000
ROWS_PT = ACC_ROWS // 16
ZROWS = 56
NM_ALL = 2 * 16 * NMAC

_MESH = plsc.VectorSubcoreMesh(
    core_axis_name="c", subcore_axis_name="s", num_cores=2, num_subcores=16)
_SDS = jax.ShapeDtypeStruct


def _zero_zbuf(zbuf):
    def zrow(i, _):
        zbuf[i, pl.ds(0, 16)] = jnp.zeros((16,), jnp.float32)
        zbuf[i, pl.ds(16, 16)] = jnp.zeros((16,), jnp.float32)
        return 0
    lax.fori_loop(0, ZROWS, zrow, 0)


def _zero_acc(acc, zbuf, s):
    def zblk(k, _):
        pltpu.sync_copy(zbuf, acc.at[pl.ds(s * ROWS_PT + k * ZROWS, ZROWS)])
        return 0
    lax.fori_loop(0, ROWS_PT // ZROWS, zblk, 0)


def _drain(acc, outP, outE, c, s):
    @pl.when((c == 0) & (s < 15))
    def _():
        pltpu.sync_copy(acc.at[pl.ds(s * 3128, 3128)],
                        outP.at[pl.ds(s * 3128, 3128)])

    @pl.when((c == 0) & (s == 15))
    def _():
        pltpu.sync_copy(acc.at[pl.ds(15 * 3128, 3080)],
                        outP.at[pl.ds(15 * 3128, 3080)])

    @pl.when((c == 1) & (s < 15))
    def _():
        pltpu.sync_copy(acc.at[pl.ds(s * 1720, 1720)],
                        outE.at[pl.ds(s * 1720, 1720)])

    @pl.when((c == 1) & (s == 15))
    def _():
        pltpu.sync_copy(acc.at[pl.ds(15 * 1720, 1704)],
                        outE.at[pl.ds(15 * 1720, 1704)])


def _make_segsum():
    scratch = [
        pltpu.VMEM((2, 2, 128), jnp.int32),
        pltpu.VMEM((2, 2, 128), jnp.int32),
        pltpu.VMEM((2, 2, 128), jnp.int32),
        pltpu.VMEM((2, 128, 32), jnp.float32),
        pltpu.VMEM((2, 128, 32), jnp.float32),
        pltpu.VMEM((2, 128, 32), jnp.float32),
        pltpu.VMEM_SHARED((ACC_ROWS, 32), jnp.float32),
        pltpu.VMEM((ZROWS, 32), jnp.float32),
        pltpu.SemaphoreType.DMA,
        pltpu.SemaphoreType.DMA,
        pltpu.SemaphoreType.DMA,
    ]

    def body(tableP, tableE, esrc, edst, zd, outP, outE, mbuf0, mbuf1,
             mbuf2, rows0, rows1, rows2, acc, zbuf, isem, gsem, ssem):
        c = lax.axis_index("c")
        s = lax.axis_index("s")
        mbufs = (mbuf0, mbuf1, mbuf2)
        rows = (rows0, rows1, rows2)
        _zero_zbuf(zbuf)
        _zero_acc(acc, zbuf, s)
        plsc.subcore_barrier()
        mbase = (c * 16 + s) * NMAC
        pltpu.sync_copy(esrc.at[mbase], mbuf0.at[0])
        pltpu.sync_copy(edst.at[mbase], mbuf0.at[1])

        def step(g, p):
            mb, rw = mbufs[p], rows[p]
            omb, orw = mbufs[(p + 2) % 3], rows[(p + 2) % 3]

            @pl.when(g >= 2)
            def _():
                pltpu.make_async_copy(zd, rows[(p + 1) % 3], ssem).wait()

            @pl.when(g >= 1)
            def _():
                pltpu.make_async_copy(esrc.at[0], mb.at[0], isem).wait()
                pltpu.make_async_copy(edst.at[0], mb.at[1], isem).wait()

            @pl.when(g < NMAC - 1)
            def _():
                pltpu.async_copy(esrc.at[mbase + g + 1],
                                 mbufs[(p + 1) % 3].at[0], isem)
                pltpu.async_copy(edst.at[mbase + g + 1],
                                 mbufs[(p + 1) % 3].at[1], isem)

            @pl.when(c == 0)
            def _():
                for j in range(2):
                    pltpu.async_copy(tableE.at[mb.at[0, j]], rw.at[j], gsem)

            @pl.when(c == 1)
            def _():
                for j in range(2):
                    pltpu.async_copy(tableP.at[mb.at[0, j]], rw.at[j], gsem)

            @pl.when(g >= 1)
            def _():
                pltpu.make_async_copy(zd, orw, gsem).wait()
                for j in range(2):
                    pltpu.async_copy(orw.at[j], acc.at[omb.at[1, j]],
                                     ssem, add=True)

        def chunk(g, _):
            for p in range(3):
                @pl.when(g % 3 == p)
                def _(p=p):
                    step(g, p)
            return 0

        lax.fori_loop(0, NMAC, chunk, 0)
        pltpu.make_async_copy(zd, rows0, gsem).wait()
        for j in range(2):
            pltpu.async_copy(rows0.at[j], acc.at[mbuf0.at[1, j]], ssem,
                             add=True)
        pltpu.make_async_copy(zd, rows2, ssem).wait()
        pltpu.make_async_copy(zd, rows0, ssem).wait()
        plsc.subcore_barrier()
        _drain(acc, outP, outE, c, s)

    return pl.kernel(
        body,
        out_type=(_SDS((NP_, 32), jnp.float32),
                  _SDS((NEPAD, 32), jnp.float32)),
        mesh=_MESH,
        compiler_params=pltpu.CompilerParams(use_tc_tiling_on_sc=False),
        scratch_types=scratch,
    )


def _make_degree():
    scratch = [
        pltpu.VMEM((2, 128), jnp.int32),
        pltpu.VMEM((2, 128), jnp.int32),
        pltpu.VMEM((2, 128, 16), jnp.float32),
        pltpu.VMEM_SHARED((ACC_ROWS, 16), jnp.float32),
        pltpu.VMEM((ZROWS, 16), jnp.float32),
        pltpu.SemaphoreType.DMA,
    ]

    def body(edst, zd16, outP, outE, dbufA, dbufB, ones16, acc, zbuf16,
             dsem):
        c = lax.axis_index("c")
        s = lax.axis_index("s")

        def zrow(i, _):
            zbuf16[i, pl.ds(0, 16)] = jnp.zeros((16,), jnp.float32)
            return 0
        lax.fori_loop(0, ZROWS, zrow, 0)

        def orow(i, _):
            ones16[i // 128, i % 128, pl.ds(0, 16)] = \
                jnp.ones((16,), jnp.float32)
            return 0
        lax.fori_loop(0, 256, orow, 0)
        _zero_acc(acc, zbuf16, s)
        plsc.subcore_barrier()
        mbase = (c * 16 + s) * NMAC

        def step(g, dbuf):
            @pl.when(g >= 2)
            def _():
                pltpu.make_async_copy(zd16, ones16, dsem).wait()

            pltpu.sync_copy(edst.at[mbase + g], dbuf)
            for j in range(2):
                pltpu.async_copy(ones16.at[j], acc.at[dbuf.at[j]], dsem,
                                 add=True)

        def chunk(g, _):
            @pl.when(g % 2 == 0)
            def _():
                step(g, dbufA)

            @pl.when(g % 2 == 1)
            def _():
                step(g, dbufB)
            return 0

        lax.fori_loop(0, NMAC, chunk, 0)
        pltpu.make_async_copy(zd16, ones16, dsem).wait()
        pltpu.make_async_copy(zd16, ones16, dsem).wait()
        plsc.subcore_barrier()
        _drain(acc, outP, outE, c, s)

    return pl.kernel(
        body,
        out_type=(_SDS((NP_, 16), jnp.float32),
                  _SDS((NEPAD, 16), jnp.float32)),
        mesh=_MESH,
        compiler_params=pltpu.CompilerParams(use_tc_tiling_on_sc=False),
        scratch_types=scratch,
    )


_segsum = _make_segsum()
_sc_degree = _make_degree()


def _k1_body(xp, wp, bp, out):
    out[...] = jax.nn.relu(
        jnp.dot(xp[...], wp[...], preferred_element_type=jnp.float32)
        + bp[...])


def _tc_prod(x_product, Wp, bp):
    R, G = 2000, 25
    return pl.pallas_call(
        _k1_body,
        grid=(G,),
        in_specs=[
            pl.BlockSpec((R, DIN_), lambda i: (i, 0)),
            pl.BlockSpec((DIN_, H_), lambda i: (0, 0)),
            pl.BlockSpec((1, H_), lambda i: (0, 0)),
        ],
        out_specs=pl.BlockSpec((R, H_), lambda i: (i, 0)),
        out_shape=_SDS((NP_, H_), jnp.float32),
    )(x_product, Wp, bp.reshape(1, H_))


def _k2_body(x, w1l, w1r, b1, outa, outb, outr):
    xb = x[...]
    xl = jnp.dot(xb, w1l[...], preferred_element_type=jnp.float32)
    outa[...] = xl[:, :32]
    outb[...] = xl[:, 32:]
    outr[...] = jnp.dot(xb, w1r[...], preferred_element_type=jnp.float32) \
        + b1[...]


def _tc_lin1(x, W1l, W1r, b1, R, G):
    n = R * G
    return pl.pallas_call(
        _k2_body,
        grid=(G,),
        in_specs=[
            pl.BlockSpec((R, H_), lambda i: (i, 0)),
            pl.BlockSpec((H_, H_), lambda i: (0, 0)),
            pl.BlockSpec((H_, H_), lambda i: (0, 0)),
            pl.BlockSpec((1, H_), lambda i: (0, 0)),
        ],
        out_specs=[
            pl.BlockSpec((R, 32), lambda i: (i, 0)),
            pl.BlockSpec((R, 32), lambda i: (i, 0)),
            pl.BlockSpec((R, H_), lambda i: (i, 0)),
        ],
        out_shape=[
            _SDS((n, 32), jnp.float32),
            _SDS((n, 32), jnp.float32),
            _SDS((n, H_), jnp.float32),
        ],
    )(x, W1l, W1r, b1.reshape(1, H_))


def _k3_body(agga, aggb, deg, xrb, w2l, w2r, b2, outl, outr):
    inv = 1.0 / jnp.maximum(deg[...][:, :1], 1.0)
    h = jax.nn.relu(
        jnp.concatenate([agga[...] * inv, aggb[...] * inv], axis=1)
        + xrb[...])
    outl[...] = jnp.dot(h, w2l[...], preferred_element_type=jnp.float32)
    outr[...] = jnp.dot(h, w2r[...], preferred_element_type=jnp.float32) \
        + b2[...]


def _tc_layer2in(aggA, aggB, deg16, xrb, W2l, W2r, b2, R, G):
    n = R * G
    return pl.pallas_call(
        _k3_body,
        grid=(G,),
        in_specs=[
            pl.BlockSpec((R, 32), lambda i: (i, 0)),
            pl.BlockSpec((R, 32), lambda i: (i, 0)),
            pl.BlockSpec((R, 16), lambda i: (i, 0)),
            pl.BlockSpec((R, H_), lambda i: (i, 0)),
            pl.BlockSpec((H_, OUT_), lambda i: (0, 0)),
            pl.BlockSpec((H_, OUT_), lambda i: (0, 0)),
            pl.BlockSpec((1, OUT_), lambda i: (0, 0)),
        ],
        out_specs=[
            pl.BlockSpec((R, OUT_), lambda i: (i, 0)),
            pl.BlockSpec((R, OUT_), lambda i: (i, 0)),
        ],
        out_shape=[
            _SDS((n, OUT_), jnp.float32),
            _SDS((n, OUT_), jnp.float32),
        ],
    )(aggA, aggB, deg16, xrb, W2l, W2r, b2.reshape(1, OUT_))


def _k4_body(agg2, deg, hrb, out):
    inv = 1.0 / jnp.maximum(deg[...][:, :1], 1.0)
    out[...] = agg2[...] * inv + hrb[...]


def _tc_final(agg2, deg16, hrb2, R, G):
    return pl.pallas_call(
        _k4_body,
        grid=(G,),
        in_specs=[
            pl.BlockSpec((R, OUT_), lambda i: (i, 0)),
            pl.BlockSpec((R, 16), lambda i: (i, 0)),
            pl.BlockSpec((R, OUT_), lambda i: (i, 0)),
        ],
        out_specs=pl.BlockSpec((R, OUT_), lambda i: (i, 0)),
        out_shape=_SDS((R * G, OUT_), jnp.float32),
    )(agg2, deg16, hrb2)


def _edges(edge_pb, edge_pc, edge_ps, edge_up):
    i32 = jnp.int32
    npad = E_SIDE - E_REAL
    padz = jnp.zeros((npad,), i32)
    padt = jnp.full((npad,), TRASH, i32)
    srcA = jnp.concatenate([
        edge_pb[1] + NU_, edge_pc[1] + (NU_ + NB_),
        edge_ps[1] + (NU_ + NB_ + NC_), edge_up[0], padz])
    dstA = jnp.concatenate([
        edge_pb[0], edge_pc[0], edge_ps[0], edge_up[1], padt])
    srcB = jnp.concatenate([
        edge_pb[0], edge_pc[0], edge_ps[0], edge_up[1], padz])
    dstB = jnp.concatenate([
        edge_pb[1] + NU_, edge_pc[1] + (NU_ + NB_),
        edge_ps[1] + (NU_ + NB_ + NC_), edge_up[0], padt])
    sr = jnp.concatenate([srcA, srcB]).astype(i32).reshape(NM_ALL, 2, 128)
    dr = jnp.concatenate([dstA, dstB]).astype(i32).reshape(NM_ALL, 2, 128)
    return sr, dr


def kernel(x_product, edge_pb, edge_pc, edge_ps, edge_up, user_emb,
           brand_emb, category_emb, shop_emb, Wp, bp, W1l, W1r, b1,
           W2l, W2r, b2):
    esrc, edst = _edges(edge_pb, edge_pc, edge_ps, edge_up)
    (esrc, edst, x_product, user_emb, brand_emb, category_emb, shop_emb) = \
        lax.optimization_barrier(
            (esrc, edst, x_product, user_emb, brand_emb, category_emb,
             shop_emb))
    zd = jnp.zeros((2, 128, 32), jnp.float32)
    zd16 = jnp.zeros((2, 128, 16), jnp.float32)
    degP, degE = _sc_degree(edst, zd16)
    degP, degE, zd = lax.optimization_barrier((degP, degE, zd))

    xE = jnp.concatenate([user_emb, brand_emb, category_emb, shop_emb,
                          jnp.zeros((NEPAD - NE_, H_), jnp.float32)], axis=0)
    xlAE, xlBE, xrbE = _tc_lin1(xE, W1l, W1r, b1, 3056, 9)

    prod = _tc_prod(x_product, Wp, bp)
    xlAP, xlBP, xrbP = _tc_lin1(prod, W1l, W1r, b1, 2000, 25)

    aggAP, aggAE = _segsum(xlAP, xlAE, esrc, edst, zd)
    aggBP, aggBE = _segsum(xlBP, xlBE, esrc, edst, zd)
    hlP, hrbP = _tc_layer2in(aggAP, aggBP, degP, xrbP, W2l, W2r, b2,
                             2000, 25)
    hlE, hrbE = _tc_layer2in(aggAE, aggBE, degE, xrbE, W2l, W2r, b2,
                             3056, 9)
    agg2P, agg2E = _segsum(hlP, hlE, esrc, edst, zd)
    outP = _tc_final(agg2P, degP, hrbP, 2000, 25)
    outE = _tc_final(agg2E, degE, hrbE, 3056, 9)
    return (outP, outE[:NU_], outE[NU_:NU_ + NB_],
            outE[NU_ + NB_:NU_ + NB_ + NC_],
            outE[NU_ + NB_ + NC_:NE_])
